# Initial kernel scaffold; baseline (speedup 1.0000x reference)
import jax
import jax.numpy as jnp
from jax.experimental import pallas as pl


def kernel(x, edge_index, W_init, b_init, eps, bn_gamma, bn_beta, W_fc, b_fc, W_pred, b_pred):
    raise NotImplementedError("write your pallas kernel here")


# speedup vs baseline: 3.6856x; 3.6856x over previous
import functools

import jax
import jax.numpy as jnp
from jax import lax
from jax.experimental import pallas as pl
from jax.experimental.pallas import tpu as pltpu
from jax.experimental.pallas import tpu_sc as plsc


def _make_sc_agg(n_nodes: int, n_edges: int):
    CH = 80
    NTILES = 16
    CPT = n_edges // (NTILES * CH)
    assert CPT * NTILES * CH == n_edges
    NPT = ((n_nodes // NTILES + 7) // 8) * 8
    NROWS = NPT * NTILES
    LAST = n_nodes - NPT * (NTILES - 1)
    assert LAST > 0 and LAST % 8 == 0

    mesh = plsc.VectorSubcoreMesh(core_axis_name="c", subcore_axis_name="s")

    @functools.partial(
        pl.kernel,
        mesh=mesh,
        out_type=jax.ShapeDtypeStruct((4, n_nodes, 128), jnp.float32),
        scratch_types=[
            pltpu.VMEM((CPT, CH), jnp.int32),
            pltpu.VMEM((CPT, CH), jnp.int32),
            pltpu.VMEM((CH, 128), jnp.float32),
            pltpu.VMEM((8, 128), jnp.float32),
            pltpu.VMEM_SHARED((NROWS, 128), jnp.float32),
            pltpu.SemaphoreType.DMA,
        ],
    )
    def sc_agg(h_hbm, src_hbm, dst_hbm, out_hbm,
               srcv, dstv, rows, zbuf, aggsh, sem):
        cid = lax.axis_index("c")
        sid = lax.axis_index("s")
        row_base = pl.multiple_of(sid * NPT, 8)

        pltpu.sync_copy(src_hbm.at[sid], srcv)
        pltpu.sync_copy(dst_hbm.at[sid], dstv)

        def zb(j, carry):
            for k in range(8):
                zbuf[j, pl.ds(k * 16, 16)] = jnp.zeros((16,), jnp.float32)
            return carry
        lax.fori_loop(0, 8, zb, 0)

        def copy_rows(nrows, src_fn, dst_fn):
            full, rem = divmod(nrows, 80)
            for t in range(full):
                pltpu.sync_copy(src_fn(t * 80, 80), dst_fn(t * 80, 80))
            if rem:
                pltpu.sync_copy(src_fn(full * 80, rem), dst_fn(full * 80, rem))

        def do_chunk(f):
            def zero_body(j, carry):
                pltpu.sync_copy(
                    zbuf,
                    aggsh.at[pl.ds(pl.multiple_of(row_base + j * 8, 8), 8)])
                return carry
            lax.fori_loop(0, NPT // 8, zero_body, 0)
            plsc.subcore_barrier()

            def acc(j, carry):
                pltpu.async_copy(h_hbm.at[f].at[srcv.at[j]], rows, sem).wait()
                pltpu.sync_copy(rows, aggsh.at[dstv.at[j]], add=True)
                return carry
            lax.fori_loop(0, CPT, acc, 0)
            plsc.subcore_barrier()

            def wb(nrows):
                copy_rows(
                    nrows,
                    lambda o, k: aggsh.at[pl.ds(row_base + o, k)],
                    lambda o, k: out_hbm.at[f].at[pl.ds(row_base + o, k)],
                )

            @pl.when(sid < NTILES - 1)
            def _():
                wb(NPT)

            @pl.when(sid == NTILES - 1)
            def _():
                wb(LAST)

        @pl.when(cid == 0)
        def _():
            do_chunk(0)
            do_chunk(1)

        @pl.when(cid == 1)
        def _():
            do_chunk(2)
            do_chunk(3)

    return sc_agg


def _init_body(x_ref, w_ref, b_ref, out_ref):
    h = jnp.dot(x_ref[...], w_ref[...], preferred_element_type=jnp.float32)
    h = h + b_ref[...]
    for f in range(4):
        out_ref[f] = h[:, 128 * f:128 * (f + 1)]


def _stats_body(ep_ref, h_ref, a_ref, stat_ref):
    u = ep_ref[...] * h_ref[...] + a_ref[...]
    s = jnp.sum(u, axis=1)
    q = jnp.sum(u * u, axis=1)

    @pl.when(pl.program_id(0) == 0)
    def _():
        stat_ref[...] = jnp.zeros_like(stat_ref)

    stat_ref[:, 0, :] += s
    stat_ref[:, 1, :] += q


def _norm_body(n_nodes, last, ep_ref, stat_ref, g_ref, bt_ref, h_ref, a_ref,
               out_ref, gacc_ref):
    u = ep_ref[...] * h_ref[...] + a_ref[...]
    mean = stat_ref[:, 0:1, :] / n_nodes
    var = stat_ref[:, 1:2, :] / n_nodes - mean * mean
    scale = g_ref[...] * lax.rsqrt(var + 1e-5)
    r = jnp.maximum((u - mean) * scale + bt_ref[...], 0.0)
    out_ref[...] = r

    if last:
        @pl.when(pl.program_id(0) == 0)
        def _():
            gacc_ref[...] = jnp.zeros_like(gacc_ref)
        gacc_ref[:, 0, :] += jnp.sum(r, axis=1)


def _head_body(gp_ref, w0_ref, b0_ref, w1_ref, b1_ref, wp_ref, bp_ref,
               out_ref):
    g = jnp.sum(gp_ref[...], axis=1)
    acc = jnp.zeros((1, w0_ref.shape[2]), jnp.float32)
    for f in range(4):
        acc = acc + jnp.dot(g[f:f + 1, :], w0_ref[f],
                            preferred_element_type=jnp.float32)
    h1 = jnp.maximum(acc + b0_ref[...], 0.0)
    h2 = jnp.maximum(jnp.dot(h1, w1_ref[...],
                             preferred_element_type=jnp.float32) + b1_ref[...],
                     0.0)
    out_ref[...] = jnp.dot(h2, wp_ref[...],
                           preferred_element_type=jnp.float32) + bp_ref[...]


def kernel(x, edge_index, W_init, b_init, eps, bn_gamma, bn_beta,
           W_fc, b_fc, W_pred, b_pred):
    n, d_in = x.shape
    h_dim = W_init.shape[1]
    n_layers = eps.shape[0]
    n_edges = edge_index.shape[1]
    n_out = W_pred.shape[1]
    assert h_dim == 512

    CH = 80
    cpt = n_edges // (16 * CH)
    src2d = edge_index[0].reshape(16, cpt, CH)
    dst2d = edge_index[1].reshape(16, cpt, CH)

    BN = 1000
    nblk = n // BN

    sc_agg = _make_sc_agg(n, n_edges)

    init_call = pl.pallas_call(
        _init_body,
        grid=(nblk,),
        in_specs=[
            pl.BlockSpec((BN, d_in), lambda i: (i, 0)),
            pl.BlockSpec((d_in, h_dim), lambda i: (0, 0)),
            pl.BlockSpec((1, h_dim), lambda i: (0, 0)),
        ],
        out_specs=pl.BlockSpec((4, BN, 128), lambda i: (0, i, 0)),
        out_shape=jax.ShapeDtypeStruct((4, n, 128), jnp.float32),
    )

    stats_call = pl.pallas_call(
        _stats_body,
        grid=(nblk,),
        in_specs=[
            pl.BlockSpec((1, 1), lambda i: (0, 0)),
            pl.BlockSpec((4, BN, 128), lambda i: (0, i, 0)),
            pl.BlockSpec((4, BN, 128), lambda i: (0, i, 0)),
        ],
        out_specs=pl.BlockSpec((4, 8, 128), lambda i: (0, 0, 0)),
        out_shape=jax.ShapeDtypeStruct((4, 8, 128), jnp.float32),
    )

    def norm_call(last):
        return pl.pallas_call(
            functools.partial(_norm_body, float(n), last),
            grid=(nblk,),
            in_specs=[
                pl.BlockSpec((1, 1), lambda i: (0, 0)),
                pl.BlockSpec((4, 8, 128), lambda i: (0, 0, 0)),
                pl.BlockSpec((4, 1, 128), lambda i: (0, 0, 0)),
                pl.BlockSpec((4, 1, 128), lambda i: (0, 0, 0)),
                pl.BlockSpec((4, BN, 128), lambda i: (0, i, 0)),
                pl.BlockSpec((4, BN, 128), lambda i: (0, i, 0)),
            ],
            out_specs=[
                pl.BlockSpec((4, BN, 128), lambda i: (0, i, 0)),
                pl.BlockSpec((4, 8, 128), lambda i: (0, 0, 0)),
            ],
            out_shape=[
                jax.ShapeDtypeStruct((4, n, 128), jnp.float32),
                jax.ShapeDtypeStruct((4, 8, 128), jnp.float32),
            ],
        )

    head_call = pl.pallas_call(
        _head_body,
        in_specs=[
            pl.BlockSpec((4, 8, 128), lambda: (0, 0, 0)),
            pl.BlockSpec((4, 128, h_dim), lambda: (0, 0, 0)),
            pl.BlockSpec((1, h_dim), lambda: (0, 0)),
            pl.BlockSpec((h_dim, h_dim), lambda: (0, 0)),
            pl.BlockSpec((1, h_dim), lambda: (0, 0)),
            pl.BlockSpec((h_dim, n_out), lambda: (0, 0)),
            pl.BlockSpec((1, n_out), lambda: (0, 0)),
        ],
        out_specs=pl.BlockSpec((1, n_out), lambda: (0, 0)),
        out_shape=jax.ShapeDtypeStruct((1, n_out), jnp.float32),
    )

    epsp1 = (1.0 + eps).reshape(n_layers, 1, 1)
    gc = bn_gamma.reshape(n_layers, 4, 1, 128)
    bc = bn_beta.reshape(n_layers, 4, 1, 128)

    hc = init_call(x, W_init, b_init.reshape(1, h_dim))
    gacc = None
    for l in range(n_layers):
        agg = sc_agg(hc, src2d, dst2d)
        stats = stats_call(epsp1[l], hc, agg)
        hc, gacc = norm_call(l == n_layers - 1)(
            epsp1[l], stats, gc[l], bc[l], hc, agg)

    out = head_call(
        gacc,
        W_fc[0].reshape(4, 128, h_dim),
        b_fc[0].reshape(1, h_dim),
        W_fc[1],
        b_fc[1].reshape(1, h_dim),
        W_pred,
        b_pred.reshape(1, n_out),
    )
    return out.reshape(n_out)
